# baseline (speedup 1.0000x reference)
import jax
import jax.numpy as jnp
from jax.experimental import pallas as pl
from jax.experimental.pallas import tpu as pltpu


def _body(lg_ref, bxt_ref, ci_ref, lab_ref, boxt_ref, cnt_ref):
    i = pl.program_id(0)
    x = lg_ref[...]
    br, c = x.shape
    m = jnp.max(x, axis=-1, keepdims=True)
    ci = jnp.broadcast_to(ci_ref[...], x.shape)
    a = jnp.min(jnp.where(x == m, ci, c), axis=-1, keepdims=True)
    lab_col = jnp.where(m > 0.0, a, -1)
    rows = []
    for k in range(br // 128):
        bc = jnp.broadcast_to(lab_col[k * 128:(k + 1) * 128, :], (128, 128))
        rows.append(bc.T[0:1, :])
    lab_lane = jnp.concatenate(rows, axis=0)
    valid = lab_lane >= 0
    lab_ref[...] = lab_lane
    boxt_ref[...] = jnp.where(valid[None], bxt_ref[...], 0.0)

    @pl.when(i == 0)
    def _():
        cnt_ref[0, 0] = 0.0

    cnt_ref[0, 0] += jnp.sum(valid.astype(jnp.float32))


def kernel(pred_logits, pred_boxes):
    B, Q, C = pred_logits.shape
    R = B * Q
    lg = pred_logits.reshape(R, C)
    bxt = pred_boxes.reshape(R, 4).T.reshape(4, R // 128, 128)
    cidx = jnp.arange(C, dtype=jnp.int32).reshape(1, C)

    BR = 4096
    BL = BR // 128
    labels, boxest, cnt = pl.pallas_call(
        _body,
        grid=(R // BR,),
        in_specs=[
            pl.BlockSpec((BR, C), lambda i: (i, 0)),
            pl.BlockSpec((4, BL, 128), lambda i: (0, i, 0)),
            pl.BlockSpec((1, C), lambda i: (0, 0)),
        ],
        out_specs=[
            pl.BlockSpec((BL, 128), lambda i: (i, 0)),
            pl.BlockSpec((4, BL, 128), lambda i: (0, i, 0)),
            pl.BlockSpec((1, 1), lambda i: (0, 0), memory_space=pltpu.SMEM),
        ],
        out_shape=[
            jax.ShapeDtypeStruct((R // 128, 128), jnp.int32),
            jax.ShapeDtypeStruct((4, R // 128, 128), jnp.float32),
            jax.ShapeDtypeStruct((1, 1), jnp.float32),
        ],
    )(lg, bxt, cidx)
    num_boxes = jnp.maximum(cnt[0, 0], 1.0)
    boxes = boxest.reshape(4, R).T.reshape(B, Q, 4)
    return labels.reshape(B, Q), boxes, num_boxes


# speedup vs baseline: 1.0588x; 1.0588x over previous
import jax
import jax.numpy as jnp
from jax.experimental import pallas as pl
from jax.experimental.pallas import tpu as pltpu


def _body(lg_ref, bxt_ref, ci_ref, lab_ref, boxt_ref):
    x = lg_ref[...]
    br, c = x.shape
    m = jnp.max(x, axis=-1, keepdims=True)
    ci = jnp.broadcast_to(ci_ref[...], x.shape)
    a = jnp.min(jnp.where(x == m, ci, c), axis=-1, keepdims=True)
    lab_col = jnp.where(m > 0.0, a, -1)
    rows = []
    for k in range(br // 128):
        bc = jnp.broadcast_to(lab_col[k * 128:(k + 1) * 128, :], (128, 128))
        rows.append(bc.T[0:1, :])
    lab_lane = jnp.concatenate(rows, axis=0)
    valid = lab_lane >= 0
    lab_ref[...] = lab_lane
    boxt_ref[...] = jnp.where(valid[None], bxt_ref[...], 0.0)


def kernel(pred_logits, pred_boxes):
    B, Q, C = pred_logits.shape
    R = B * Q
    lg = pred_logits.reshape(R, C)
    bxt = pred_boxes.reshape(R, 4).T.reshape(4, R // 128, 128)
    cidx = jnp.arange(C, dtype=jnp.int32).reshape(1, C)

    BR = 8192
    BL = BR // 128
    labels, boxest = pl.pallas_call(
        _body,
        grid=(R // BR,),
        in_specs=[
            pl.BlockSpec((BR, C), lambda i: (i, 0)),
            pl.BlockSpec((4, BL, 128), lambda i: (0, i, 0)),
            pl.BlockSpec((1, C), lambda i: (0, 0)),
        ],
        out_specs=[
            pl.BlockSpec((BL, 128), lambda i: (i, 0)),
            pl.BlockSpec((4, BL, 128), lambda i: (0, i, 0)),
        ],
        out_shape=[
            jax.ShapeDtypeStruct((R // 128, 128), jnp.int32),
            jax.ShapeDtypeStruct((4, R // 128, 128), jnp.float32),
        ],
    )(lg, bxt, cidx)
    labels2 = labels.reshape(R)
    num_boxes = jnp.maximum(jnp.sum(labels2 >= 0).astype(jnp.float32), 1.0)
    boxes = boxest.reshape(4, R).T.reshape(B, Q, 4)
    return labels2.reshape(B, Q), boxes, num_boxes
